# baseline (speedup 1.0000x reference)
import functools

import jax
import jax.numpy as jnp
import numpy as np
from jax import lax
from jax.experimental import pallas as pl
from jax.experimental.pallas import tpu as pltpu
from jax.experimental.pallas import tpu_sc as plsc

_EPS = 1e-7
_SIGMA_BOOST = 2.0
_B, _CIN, _H, _W = 2, 96, 56, 56
_COUT = 384
_K = 9
_GADD, _RADD = 2, 2
_REGION = (max(int(0.2 * _H), 2), max(int(0.2 * _W), 2))
_MIN_SIGMA = 0.05
_SIGMA_SCALE = 0.05
_MMULT = 0.1
_VS = 4 + _GADD + _RADD
_HW = _H * _W


def _coordinates(h, w):
    r = jnp.arange(h, dtype=jnp.float32) / (h - 1)
    c = jnp.arange(w, dtype=jnp.float32) / (w - 1)
    rr = jnp.broadcast_to(r[:, None], (h, w))
    cc = jnp.broadcast_to(c[None, :], (h, w))
    return jnp.stack([rr, cc], axis=0)


def _inv(x, mx):
    sc = (x / mx) * 0.9999 + 0.00005
    return jnp.log(sc / (1.0 - sc))


def _rand_uniforms(b, h, w, k):
    skey = jax.random.key(42)
    k1, k2 = jax.random.split(skey)
    u1 = jax.random.uniform(k1, (b, h, w, k, _GADD, 2))
    u2 = jax.random.uniform(k2, (b, h, w, k, _RADD, 2))
    return u1, u2


def _rand_lanes(b, h, w, k):
    u1, u2 = _rand_uniforms(b, h, w, k)
    z = jnp.zeros((b, h, w, k), jnp.float32)
    igr = jnp.stack(
        [z, z, z, z,
         jnp.floor(u1[..., 0, 0] * h), jnp.floor(u1[..., 1, 0] * h),
         u2[..., 0, 0] * _REGION[0], u2[..., 1, 0] * _REGION[0]],
        axis=-1).reshape(b * h * w, k * _VS)
    jgr = jnp.stack(
        [z, z, z, z,
         jnp.floor(u1[..., 0, 1] * w), jnp.floor(u1[..., 1, 1] * w),
         u2[..., 0, 1] * _REGION[1], u2[..., 1, 1] * _REGION[1]],
        axis=-1).reshape(b * h * w, k * _VS)
    return igr, jgr


_NTILES = 32
_CP = 4
_PIX_PER_TILE = (_B * _HW) // _NTILES
_CHUNKS = _PIX_PER_TILE // _CP
_KVS = _K * _VS
_KC = _K * _CIN


def _full16(v):
    return jnp.full((16,), v, dtype=jnp.int32)


def _sc_gather_combine(lin, wts, table):
    mesh = plsc.VectorSubcoreMesh(core_axis_name="c", subcore_axis_name="s")

    @functools.partial(
        pl.kernel,
        mesh=mesh,
        out_type=jax.ShapeDtypeStruct((_B * _HW, _KC), jnp.float32),
        scratch_types=[
            pltpu.VMEM((_CP, _KVS), jnp.int32),
            pltpu.VMEM((_CP, _KVS), jnp.int32),
            pltpu.VMEM((_CP * _KVS + 16,), jnp.float32),
            pltpu.VMEM((_CP * _KVS + 16,), jnp.float32),
            pltpu.VMEM((_CP, _KVS, _CIN), jnp.float32),
            pltpu.VMEM((_CP, _KVS, _CIN), jnp.float32),
            pltpu.VMEM((_CP, _KC), jnp.float32),
            pltpu.SemaphoreType.DMA,
            pltpu.SemaphoreType.DMA,
        ],
        compiler_params=pltpu.CompilerParams(use_tc_tiling_on_sc=False),
    )
    def body(lin_hbm, w_hbm, table_hbm, out_hbm, idx0, idx1, wv0, wv1,
             rows0, rows1, out_v, sem0, sem1):
        wid = lax.axis_index("s") * 2 + lax.axis_index("c")
        base = wid * _PIX_PER_TILE
        sems = (sem0, sem1)
        idxs = (idx0, idx1)
        wvs = (wv0, wv1)
        rows = (rows0, rows1)

        def start_chunk(i, par):
            p0 = base + i * _CP
            pltpu.sync_copy(lin_hbm.at[pl.ds(p0, _CP)], idxs[par])
            pltpu.sync_copy(w_hbm.at[pl.ds(p0 * _KVS, _CP * _KVS)],
                            wvs[par].at[pl.ds(0, _CP * _KVS)])
            for ci in range(_CP):
                pltpu.async_copy(table_hbm.at[idxs[par].at[ci]],
                                 rows[par].at[ci], sems[par])

        def compute_chunk(i, par):
            for ci in range(_CP):
                pltpu.make_async_copy(table_hbm.at[idxs[par].at[ci]],
                                      rows[par].at[ci], sems[par]).wait()

            def k_body(k, c2):
                kv0 = k * _VS
                for ci in range(_CP):
                    acc = None
                    wv16 = wvs[par][pl.ds(ci * _KVS + kv0, 16)]
                    for v in range(_VS):
                        wvec = wv16[v]
                        terms = [
                            wvec * rows[par][ci, kv0 + v,
                                             pl.ds(j * 16, 16)]
                            for j in range(_CIN // 16)
                        ]
                        if acc is None:
                            acc = terms
                        else:
                            acc = [a + t for a, t in zip(acc, terms)]
                    for j in range(_CIN // 16):
                        out_v[ci, pl.ds(k * _CIN + j * 16, 16)] = acc[j]
                return c2

            lax.fori_loop(0, _K, k_body, 0)
            p0 = base + i * _CP
            pltpu.sync_copy(out_v, out_hbm.at[pl.ds(p0, _CP)])

        start_chunk(0, 0)

        def loop_body(i, carry):
            par = lax.rem(i, 2)

            @pl.when(jnp.logical_and(i + 1 < _CHUNKS, par == 0))
            def _():
                start_chunk(i + 1, 1)

            @pl.when(jnp.logical_and(i + 1 < _CHUNKS, par == 1))
            def _():
                start_chunk(i + 1, 0)

            @pl.when(par == 0)
            def _():
                compute_chunk(i, 0)

            @pl.when(par == 1)
            def _():
                compute_chunk(i, 1)

            return carry

        lax.fori_loop(0, _CHUNKS, loop_body, 0)

    return body(lin, wts, table)


_P1_BLK = 448


def _p1_kernel(m0_ref, m1_ref, sig_ref, igr_ref, jgr_ref,
               e_ref, a_ref, mbig_ref, bsum_ref, lin_ref, w_ref):
    f32 = jnp.float32
    hi = jax.lax.Precision.HIGHEST
    m0, m1, sig = m0_ref[...], m1_ref[...], sig_ref[...]
    fl0, fl1 = jnp.floor(m0), jnp.floor(m1)
    low0 = jnp.clip(jnp.round(m0) - 5.5, 0.0, 45.0)
    low1 = jnp.clip(jnp.round(m1) - 5.5, 0.0, 45.0)

    e = e_ref[...]

    def ex(v):
        return jnp.dot(v, e, preferred_element_type=f32, precision=hi)

    m0e, m1e = ex(m0), ex(m1)
    sige = ex(sig)
    igr, jgr = igr_ref[...], jgr_ref[...]
    lane = jax.lax.broadcasted_iota(jnp.int32, (1, 9 * _VS), 1)
    vm = lane % _VS
    mask_n = vm < 4
    mask_g = jnp.logical_or(vm == 4, vm == 5)
    i_n = ex(fl0) + a_ref[0:1, :]
    j_n = ex(fl1) + a_ref[1:2, :]
    i_r = jnp.floor(ex(low0) + igr)
    j_r = jnp.floor(ex(low1) + jgr)
    i_all = jnp.where(mask_n, i_n, jnp.where(mask_g, igr, i_r))
    j_all = jnp.where(mask_n, j_n, jnp.where(mask_g, jgr, j_r))
    i_all = jnp.clip(i_all, 0.0, 55.0)
    j_all = jnp.clip(j_all, 0.0, 55.0)
    code = i_all * 56.0 + j_all

    shifted = jnp.dot(code, mbig_ref[...], preferred_element_type=f32,
                      precision=hi)
    dup = jnp.zeros(code.shape, dtype=bool)
    for d in range(1, _VS):
        eq = code == shifted[:, (d - 1) * 72:d * 72]
        dup = jnp.logical_or(dup, jnp.logical_and(eq, vm >= d))

    di = (i_all - m0e) / (sige + _EPS)
    dj = (j_all - m1e) / (sige + _EPS)
    props = jnp.exp(-0.5 * (di * di + dj * dj))
    props = jnp.where(dup, 0.0, props)
    psum = jnp.dot(props, bsum_ref[...], preferred_element_type=f32,
                   precision=hi)
    w_ref[...] = props / psum

    boff = jnp.where(pl.program_id(0) >= (_HW // _P1_BLK), float(_HW), 0.0)
    lin_ref[...] = (code + boff).astype(jnp.int32)


def _phase1(m0, m1, sig, igr, jgr):
    f32 = jnp.float32
    e = np.zeros((9, 72), np.float32)
    a = np.zeros((2, 72), np.float32)
    for k in range(9):
        for v in range(_VS):
            e[k, k * 8 + v] = 1.0
        a[0, k * 8 + 2] = a[0, k * 8 + 3] = 1.0
        a[1, k * 8 + 1] = a[1, k * 8 + 3] = 1.0
    mbig = np.zeros((72, 7 * 72), np.float32)
    for d in range(1, _VS):
        for k in range(9):
            for v in range(d, _VS):
                mbig[k * 8 + v - d, (d - 1) * 72 + k * 8 + v] = 1.0
    bsum = np.zeros((72, 72), np.float32)
    for k in range(9):
        bsum[k * 8:k * 8 + 8, k * 8:k * 8 + 8] = 1.0

    nblk = (_B * _HW) // _P1_BLK
    full = lambda shape: pl.BlockSpec(shape, lambda i: tuple(0 for _ in shape))
    row = lambda shape: pl.BlockSpec(shape, lambda i: (i, 0))
    return pl.pallas_call(
        _p1_kernel,
        grid=(nblk,),
        in_specs=[
            row((_P1_BLK, 9)),
            row((_P1_BLK, 9)),
            row((_P1_BLK, 9)),
            row((_P1_BLK, 72)),
            row((_P1_BLK, 72)),
            full((9, 72)),
            full((2, 72)),
            full((72, 7 * 72)),
            full((72, 72)),
        ],
        out_specs=[row((_P1_BLK, 72)), row((_P1_BLK, 72))],
        out_shape=[
            jax.ShapeDtypeStruct((_B * _HW, 72), jnp.int32),
            jax.ShapeDtypeStruct((_B * _HW, 72), f32),
        ],
    )(m0, m1, sig, igr, jgr, jnp.asarray(e), jnp.asarray(a),
      jnp.asarray(mbig), jnp.asarray(bsum))


def _final_matmul_kernel(feats_ref, wu_ref, bu_ref, out_ref):
    out_ref[...] = (
        jnp.dot(feats_ref[...], wu_ref[...],
                preferred_element_type=jnp.float32)
        + bu_ref[...][None, :]
    )


def _final_matmul(feats, Wu, bu):
    bhw, kc = feats.shape
    cout = Wu.shape[1]
    blk = 448
    grid = (bhw // blk,)
    return pl.pallas_call(
        _final_matmul_kernel,
        grid=grid,
        in_specs=[
            pl.BlockSpec((blk, kc), lambda i: (i, 0)),
            pl.BlockSpec((kc, cout), lambda i: (0, 0)),
            pl.BlockSpec((cout,), lambda i: (0,)),
        ],
        out_specs=pl.BlockSpec((blk, cout), lambda i: (i, 0)),
        out_shape=jax.ShapeDtypeStruct((bhw, cout), jnp.float32),
    )(feats, Wu, bu)


def kernel(x, W1, b1, W2, b2, Wu, bu):
    b, c, h, w = x.shape
    k = _K
    hwv = jnp.array([h, w], jnp.float32)
    coords = _coordinates(h, w)
    cb = jnp.broadcast_to(coords[None, :, :, :], (b, 2, h, w))
    xin = jnp.transpose(jnp.concatenate([x, cb], axis=1), (0, 2, 3, 1))
    hdn = jax.nn.relu(xin @ W1 + b1)
    params = hdn @ W2 + b2
    mids = coords[None] * (hwv - 1.0)[None, :, None, None]
    mids = _inv(jnp.transpose(mids, (0, 2, 3, 1)), hwv[None, None, None, :])
    mids = jnp.broadcast_to(mids[:, :, :, None, :], (b, h, w, k, 2))
    means = params[..., : k * 2].reshape(b, h, w, k, 2)
    sigmas = params[..., k * 2:].reshape(b, h, w, k)
    means = mids + _MMULT * means
    means = jax.nn.sigmoid(means) * (hwv - 1.0)
    sig = (jax.nn.softplus(sigmas + _SIGMA_BOOST) + _MIN_SIGMA
           ) * hwv[0] * _SIGMA_SCALE
    m0 = means[..., 0].reshape(b * h * w, k)
    m1 = means[..., 1].reshape(b * h * w, k)
    sigp = sig.reshape(b * h * w, k)
    igr, jgr = _rand_lanes(b, h, w, k)

    lin, wts2d = _phase1(m0, m1, sigp, igr, jgr)
    wts = wts2d.reshape(-1)
    table = jnp.transpose(x, (0, 2, 3, 1)).reshape(b * h * w, c)
    feats = _sc_gather_combine(lin, wts, table)

    out = _final_matmul(feats, Wu, bu)
    out = out.reshape(b, h, w, _COUT)
    return jnp.transpose(out, (0, 3, 1, 2))


# speedup vs baseline: 1.0628x; 1.0628x over previous
import functools

import jax
import jax.numpy as jnp
import numpy as np
from jax import lax
from jax.experimental import pallas as pl
from jax.experimental.pallas import tpu as pltpu
from jax.experimental.pallas import tpu_sc as plsc

_EPS = 1e-7
_SIGMA_BOOST = 2.0
_B, _CIN, _H, _W = 2, 96, 56, 56
_COUT = 384
_K = 9
_GADD, _RADD = 2, 2
_REGION = (max(int(0.2 * _H), 2), max(int(0.2 * _W), 2))
_MIN_SIGMA = 0.05
_SIGMA_SCALE = 0.05
_MMULT = 0.1
_VS = 4 + _GADD + _RADD
_HW = _H * _W


def _coordinates(h, w):
    r = jnp.arange(h, dtype=jnp.float32) / (h - 1)
    c = jnp.arange(w, dtype=jnp.float32) / (w - 1)
    rr = jnp.broadcast_to(r[:, None], (h, w))
    cc = jnp.broadcast_to(c[None, :], (h, w))
    return jnp.stack([rr, cc], axis=0)


def _inv(x, mx):
    sc = (x / mx) * 0.9999 + 0.00005
    return jnp.log(sc / (1.0 - sc))


def _rand_uniforms(b, h, w, k):
    skey = jax.random.key(42)
    k1, k2 = jax.random.split(skey)
    u1 = jax.random.uniform(k1, (b, h, w, k, _GADD, 2))
    u2 = jax.random.uniform(k2, (b, h, w, k, _RADD, 2))
    return u1, u2


def _rand_lanes(b, h, w, k):
    u1, u2 = _rand_uniforms(b, h, w, k)
    z = jnp.zeros((b, h, w, k), jnp.float32)
    igr = jnp.stack(
        [z, z, z, z,
         jnp.floor(u1[..., 0, 0] * h), jnp.floor(u1[..., 1, 0] * h),
         u2[..., 0, 0] * _REGION[0], u2[..., 1, 0] * _REGION[0]],
        axis=-1).reshape(b * h * w, k * _VS)
    jgr = jnp.stack(
        [z, z, z, z,
         jnp.floor(u1[..., 0, 1] * w), jnp.floor(u1[..., 1, 1] * w),
         u2[..., 0, 1] * _REGION[1], u2[..., 1, 1] * _REGION[1]],
        axis=-1).reshape(b * h * w, k * _VS)
    return igr, jgr


_NTILES = 32
_CP = 4
_PIX_PER_TILE = (_B * _HW) // _NTILES
_CHUNKS = _PIX_PER_TILE // _CP
_KVS = _K * _VS
_KC = _K * _CIN


def _full16(v):
    return jnp.full((16,), v, dtype=jnp.int32)


def _sc_gather_combine(lin, wts, table):
    mesh = plsc.VectorSubcoreMesh(core_axis_name="c", subcore_axis_name="s")

    @functools.partial(
        pl.kernel,
        mesh=mesh,
        out_type=jax.ShapeDtypeStruct((_B * _HW, _KC), jnp.float32),
        scratch_types=[
            pltpu.VMEM((_CP, _KVS), jnp.int32),
            pltpu.VMEM((_CP, _KVS), jnp.int32),
            pltpu.VMEM((_CP, _KVS), jnp.int32),
            pltpu.VMEM((_CP * _KVS + 16,), jnp.float32),
            pltpu.VMEM((_CP * _KVS + 16,), jnp.float32),
            pltpu.VMEM((_CP * _KVS + 16,), jnp.float32),
            pltpu.VMEM((_CP, _KVS, 128), jnp.float32),
            pltpu.VMEM((_CP, _KVS, 128), jnp.float32),
            pltpu.VMEM((_CP, _KVS, 128), jnp.float32),
            pltpu.VMEM((_CP, _KC), jnp.float32),
            pltpu.SemaphoreType.DMA,
            pltpu.SemaphoreType.DMA,
            pltpu.SemaphoreType.DMA,
        ],
    )
    def body(lin_hbm, w_hbm, table_hbm, out_hbm, idx0, idx1, idx2,
             wv0, wv1, wv2, rows0, rows1, rows2, out_v, sem0, sem1, sem2):
        wid = lax.axis_index("s") * 2 + lax.axis_index("c")
        base = wid * _PIX_PER_TILE
        sems = (sem0, sem1, sem2)
        idxs = (idx0, idx1, idx2)
        wvs = (wv0, wv1, wv2)
        rows = (rows0, rows1, rows2)

        def start_chunk(i, par):
            p0 = base + i * _CP
            pltpu.sync_copy(lin_hbm.at[pl.ds(p0, _CP)], idxs[par])
            pltpu.sync_copy(w_hbm.at[pl.ds(p0 * _KVS, _CP * _KVS)],
                            wvs[par].at[pl.ds(0, _CP * _KVS)])
            for ci in range(_CP):
                pltpu.async_copy(table_hbm.at[idxs[par].at[ci]],
                                 rows[par].at[ci], sems[par])

        def compute_chunk(i, par):
            for ci in range(_CP):
                pltpu.make_async_copy(table_hbm.at[idxs[par].at[ci]],
                                      rows[par].at[ci], sems[par]).wait()

            def k_body(k, c2):
                kv0 = k * _VS
                for ci in range(_CP):
                    acc = None
                    wv16 = wvs[par][pl.ds(ci * _KVS + kv0, 16)]
                    for v in range(_VS):
                        wvec = wv16[v]
                        terms = [
                            wvec * rows[par][ci, kv0 + v,
                                             pl.ds(j * 16, 16)]
                            for j in range(_CIN // 16)
                        ]
                        if acc is None:
                            acc = terms
                        else:
                            acc = [a + t for a, t in zip(acc, terms)]
                    for j in range(_CIN // 16):
                        out_v[ci, pl.ds(k * _CIN + j * 16, 16)] = acc[j]
                return c2

            lax.fori_loop(0, _K, k_body, 0)
            p0 = base + i * _CP
            pltpu.sync_copy(out_v, out_hbm.at[pl.ds(p0, _CP)])

        start_chunk(0, 0)
        start_chunk(1, 1)

        def loop_body(i, carry):
            par = lax.rem(i, 3)
            for p in range(3):
                @pl.when(jnp.logical_and(i + 2 < _CHUNKS, par == p))
                def _(p=p):
                    start_chunk(i + 2, (p + 2) % 3)

                @pl.when(par == p)
                def _(p=p):
                    compute_chunk(i, p)

            return carry

        lax.fori_loop(0, _CHUNKS, loop_body, 0)

    return body(lin, wts, table)


_P1_BLK = 448


def _p1_kernel(m0_ref, m1_ref, sig_ref, igr_ref, jgr_ref,
               e_ref, a_ref, mbig_ref, bsum_ref, lin_ref, w_ref):
    f32 = jnp.float32
    hi = jax.lax.Precision.HIGHEST
    m0, m1, sig = m0_ref[...], m1_ref[...], sig_ref[...]
    fl0, fl1 = jnp.floor(m0), jnp.floor(m1)
    low0 = jnp.clip(jnp.round(m0) - 5.5, 0.0, 45.0)
    low1 = jnp.clip(jnp.round(m1) - 5.5, 0.0, 45.0)

    e = e_ref[...]

    def ex(v):
        return jnp.dot(v, e, preferred_element_type=f32, precision=hi)

    m0e, m1e = ex(m0), ex(m1)
    sige = ex(sig)
    igr, jgr = igr_ref[...], jgr_ref[...]
    lane = jax.lax.broadcasted_iota(jnp.int32, (1, 9 * _VS), 1)
    vm = lane % _VS
    mask_n = vm < 4
    mask_g = jnp.logical_or(vm == 4, vm == 5)
    i_n = ex(fl0) + a_ref[0:1, :]
    j_n = ex(fl1) + a_ref[1:2, :]
    i_r = jnp.floor(ex(low0) + igr)
    j_r = jnp.floor(ex(low1) + jgr)
    i_all = jnp.where(mask_n, i_n, jnp.where(mask_g, igr, i_r))
    j_all = jnp.where(mask_n, j_n, jnp.where(mask_g, jgr, j_r))
    i_all = jnp.clip(i_all, 0.0, 55.0)
    j_all = jnp.clip(j_all, 0.0, 55.0)
    code = i_all * 56.0 + j_all

    shifted = jnp.dot(code, mbig_ref[...], preferred_element_type=f32,
                      precision=hi)
    dup = jnp.zeros(code.shape, dtype=bool)
    for d in range(1, _VS):
        eq = code == shifted[:, (d - 1) * 72:d * 72]
        dup = jnp.logical_or(dup, jnp.logical_and(eq, vm >= d))

    di = (i_all - m0e) / (sige + _EPS)
    dj = (j_all - m1e) / (sige + _EPS)
    props = jnp.exp(-0.5 * (di * di + dj * dj))
    props = jnp.where(dup, 0.0, props)
    psum = jnp.dot(props, bsum_ref[...], preferred_element_type=f32,
                   precision=hi)
    w_ref[...] = props / psum

    boff = jnp.where(pl.program_id(0) >= (_HW // _P1_BLK), float(_HW), 0.0)
    lin_ref[...] = (code + boff).astype(jnp.int32)


def _phase1(m0, m1, sig, igr, jgr):
    f32 = jnp.float32
    e = np.zeros((9, 72), np.float32)
    a = np.zeros((2, 72), np.float32)
    for k in range(9):
        for v in range(_VS):
            e[k, k * 8 + v] = 1.0
        a[0, k * 8 + 2] = a[0, k * 8 + 3] = 1.0
        a[1, k * 8 + 1] = a[1, k * 8 + 3] = 1.0
    mbig = np.zeros((72, 7 * 72), np.float32)
    for d in range(1, _VS):
        for k in range(9):
            for v in range(d, _VS):
                mbig[k * 8 + v - d, (d - 1) * 72 + k * 8 + v] = 1.0
    bsum = np.zeros((72, 72), np.float32)
    for k in range(9):
        bsum[k * 8:k * 8 + 8, k * 8:k * 8 + 8] = 1.0

    nblk = (_B * _HW) // _P1_BLK
    full = lambda shape: pl.BlockSpec(shape, lambda i: tuple(0 for _ in shape))
    row = lambda shape: pl.BlockSpec(shape, lambda i: (i, 0))
    return pl.pallas_call(
        _p1_kernel,
        grid=(nblk,),
        in_specs=[
            row((_P1_BLK, 9)),
            row((_P1_BLK, 9)),
            row((_P1_BLK, 9)),
            row((_P1_BLK, 72)),
            row((_P1_BLK, 72)),
            full((9, 72)),
            full((2, 72)),
            full((72, 7 * 72)),
            full((72, 72)),
        ],
        out_specs=[row((_P1_BLK, 72)), row((_P1_BLK, 72))],
        out_shape=[
            jax.ShapeDtypeStruct((_B * _HW, 72), jnp.int32),
            jax.ShapeDtypeStruct((_B * _HW, 72), f32),
        ],
    )(m0, m1, sig, igr, jgr, jnp.asarray(e), jnp.asarray(a),
      jnp.asarray(mbig), jnp.asarray(bsum))


def _final_matmul_kernel(feats_ref, wu_ref, bu_ref, out_ref):
    out_ref[...] = (
        jnp.dot(feats_ref[...], wu_ref[...],
                preferred_element_type=jnp.float32)
        + bu_ref[...][None, :]
    )


def _final_matmul(feats, Wu, bu):
    bhw, kc = feats.shape
    cout = Wu.shape[1]
    blk = 448
    grid = (bhw // blk,)
    return pl.pallas_call(
        _final_matmul_kernel,
        grid=grid,
        in_specs=[
            pl.BlockSpec((blk, kc), lambda i: (i, 0)),
            pl.BlockSpec((kc, cout), lambda i: (0, 0)),
            pl.BlockSpec((cout,), lambda i: (0,)),
        ],
        out_specs=pl.BlockSpec((blk, cout), lambda i: (i, 0)),
        out_shape=jax.ShapeDtypeStruct((bhw, cout), jnp.float32),
    )(feats, Wu, bu)


def kernel(x, W1, b1, W2, b2, Wu, bu):
    b, c, h, w = x.shape
    k = _K
    hwv = jnp.array([h, w], jnp.float32)
    coords = _coordinates(h, w)
    cb = jnp.broadcast_to(coords[None, :, :, :], (b, 2, h, w))
    xin = jnp.transpose(jnp.concatenate([x, cb], axis=1), (0, 2, 3, 1))
    hdn = jax.nn.relu(xin @ W1 + b1)
    params = hdn @ W2 + b2
    mids = coords[None] * (hwv - 1.0)[None, :, None, None]
    mids = _inv(jnp.transpose(mids, (0, 2, 3, 1)), hwv[None, None, None, :])
    mids = jnp.broadcast_to(mids[:, :, :, None, :], (b, h, w, k, 2))
    means = params[..., : k * 2].reshape(b, h, w, k, 2)
    sigmas = params[..., k * 2:].reshape(b, h, w, k)
    means = mids + _MMULT * means
    means = jax.nn.sigmoid(means) * (hwv - 1.0)
    sig = (jax.nn.softplus(sigmas + _SIGMA_BOOST) + _MIN_SIGMA
           ) * hwv[0] * _SIGMA_SCALE
    m0 = means[..., 0].reshape(b * h * w, k)
    m1 = means[..., 1].reshape(b * h * w, k)
    sigp = sig.reshape(b * h * w, k)
    igr, jgr = _rand_lanes(b, h, w, k)

    lin, wts2d = _phase1(m0, m1, sigp, igr, jgr)
    wts = wts2d.reshape(-1)
    xflat = jnp.transpose(x, (0, 2, 3, 1)).reshape(b * h * w, c)
    table = jnp.pad(xflat, ((0, 0), (0, 128 - c)))
    feats = _sc_gather_combine(lin, wts, table)

    out = _final_matmul(feats, Wu, bu)
    out = out.reshape(b, h, w, _COUT)
    return jnp.transpose(out, (0, 3, 1, 2))
